# baseline (speedup 1.0000x reference)
import jax
import jax.numpy as jnp
from jax.experimental import pallas as pl
from jax.experimental.pallas import tpu as pltpu

_D = 64


def _copy_row(x_ref, w_hbm, o_hbm, sem):
    cp = pltpu.make_async_copy(w_hbm.at[pl.ds(x_ref[0], 1)], o_hbm, sem)
    cp.start()
    cp.wait()


def kernel(x, W):
    xs = jnp.asarray(x, jnp.int32).reshape((1,))
    return pl.pallas_call(
        _copy_row,
        in_specs=[
            pl.BlockSpec(memory_space=pltpu.SMEM),
            pl.BlockSpec(memory_space=pl.ANY),
        ],
        out_specs=pl.BlockSpec(memory_space=pl.ANY),
        scratch_shapes=[pltpu.SemaphoreType.DMA],
        out_shape=jax.ShapeDtypeStruct((1, _D), jnp.float32),
    )(xs, W)


# speedup vs baseline: 1.5778x; 1.5778x over previous
import jax
import jax.numpy as jnp
from jax.experimental import pallas as pl
from jax.experimental.pallas import tpu as pltpu

_D = 64
_V = 1000


def _onehot_dot(x_ref, wt_ref, o_ref):
    oh = (jax.lax.broadcasted_iota(jnp.int32, (1, _V), 1) == x_ref[0]).astype(
        jnp.float32
    )
    o_ref[...] = jax.lax.dot_general(
        oh, wt_ref[...], (((1,), (1,)), ((), ())),
        preferred_element_type=jnp.float32,
    )


def kernel(x, W):
    xs = jnp.asarray(x, jnp.int32).reshape((1,))
    WT = jnp.swapaxes(W, 0, 1)
    return pl.pallas_call(
        _onehot_dot,
        in_specs=[
            pl.BlockSpec(memory_space=pltpu.SMEM),
            pl.BlockSpec(memory_space=pltpu.VMEM),
        ],
        out_specs=pl.BlockSpec(memory_space=pltpu.VMEM),
        out_shape=jax.ShapeDtypeStruct((1, _D), jnp.float32),
    )(xs, WT)
